# baseline (speedup 1.0000x reference)
import jax
import jax.numpy as jnp
from jax import lax
from jax.experimental import pallas as pl
from jax.experimental.pallas import tpu as pltpu
from jax.experimental.pallas import tpu_sc as plsc

NC = 2
NS = 16
NW = NC * NS
L = 16
B_E = 128
NB0 = 120
NB1 = 40


def _seg_sum(h, src3, dst3, w3, np_rows):
  _, D = h.shape
  NB = src3.shape[1]
  CH = 8
  assert D % L == 0 and NB % CH == 0
  assert NB == NB0 and NB1 % CH == 0
  RPT = np_rows // NS
  assert RPT % B_E == 0

  mesh = plsc.VectorSubcoreMesh(core_axis_name="c", subcore_axis_name="s")

  def body(h_hbm, src_hbm, dst_hbm, w_hbm, out_hbm,
           src_v, dst_v, w_v, rows_a, rows_b, acc, sem_a, sem_b,
           sem_sa, sem_sb):
    cid = lax.axis_index("c")
    sid = lax.axis_index("s")
    wid = sid * NC + cid

    z16 = jnp.zeros((L,), jnp.float32)

    @pl.loop(0, B_E)
    def _(i):
      for c in range(D // L):
        rows_a[i, pl.ds(c * L, L)] = z16

    for k in range(RPT // B_E):
      pltpu.sync_copy(rows_a, acc.at[pl.ds(sid * RPT + k * B_E, B_E)])

    plsc.subcore_barrier()

    ng = jnp.where(cid == 0, NB0 // CH, NB1 // CH)

    def scale(rows, j):
      @pl.loop(0, B_E // L)
      def _(rb):
        wchunk = w_v[j, pl.ds(rb * L, L)]
        for i in range(L):
          wv = jnp.full((L,), wchunk[i])
          r = rb * L + i
          for c in range(D // L):
            sl = pl.ds(c * L, L)
            rows[r, sl] = rows[r, sl] * wv

    @pl.loop(0, ng)
    def _(g):
      pltpu.sync_copy(src_hbm.at[wid, pl.ds(g * CH, CH)], src_v)
      pltpu.sync_copy(dst_hbm.at[wid, pl.ds(g * CH, CH)], dst_v)
      pltpu.sync_copy(w_hbm.at[wid, pl.ds(g * CH, CH)], w_v)

      pltpu.async_copy(h_hbm.at[src_v.at[0]], rows_a, sem_a)
      pltpu.async_copy(h_hbm.at[src_v.at[1]], rows_b, sem_b)

      @pl.loop(0, CH, step=2)
      def _(j):
        for (buf, sem, ssem, jj) in ((rows_a, sem_a, sem_sa, j),
                                     (rows_b, sem_b, sem_sb, j + 1)):
          pltpu.make_async_copy(h_hbm.at[src_v.at[jj]], buf, sem).wait()
          scale(buf, jj)
          pltpu.async_copy(buf, acc.at[dst_v.at[jj]], ssem, add=True)

        for (buf, sem, ssem, jj) in ((rows_a, sem_a, sem_sa, j),
                                     (rows_b, sem_b, sem_sb, j + 1)):
          pltpu.make_async_copy(buf, acc.at[dst_v.at[jj]], ssem).wait()

          @pl.when(jj + 2 < CH)
          def _():
            pltpu.async_copy(h_hbm.at[src_v.at[jj + 2]], buf, sem)

    plsc.subcore_barrier()
    pltpu.sync_copy(acc.at[pl.ds(sid * RPT, RPT)],
                    out_hbm.at[cid, pl.ds(sid * RPT, RPT)])

  kern = pl.kernel(
      body,
      out_type=jax.ShapeDtypeStruct((NC, np_rows, D), jnp.float32),
      mesh=mesh,
      scratch_types=[
          pltpu.VMEM((CH, B_E), jnp.int32),
          pltpu.VMEM((CH, B_E), jnp.int32),
          pltpu.VMEM((CH, B_E), jnp.float32),
          pltpu.VMEM((B_E, D), jnp.float32),
          pltpu.VMEM((B_E, D), jnp.float32),
          pltpu.VMEM_SHARED((np_rows, D), jnp.float32),
          pltpu.SemaphoreType.DMA,
          pltpu.SemaphoreType.DMA,
          pltpu.SemaphoreType.DMA,
          pltpu.SemaphoreType.DMA,
      ],
  )
  return kern(h, src3, dst3, w3)


def _tc_layer(parts, W, b2d, block_n):
  _, N, Din = parts.shape
  Hout = W.shape[1]
  assert N % block_n == 0

  def body(p_ref, w_ref, b_ref, o_ref):
    x = p_ref[0] + p_ref[1]
    y = jnp.dot(x, w_ref[...], preferred_element_type=jnp.float32) + b_ref[...]
    o_ref[...] = jnp.maximum(y, 0.0)

  return pl.pallas_call(
      body,
      grid=(N // block_n,),
      in_specs=[
          pl.BlockSpec((NC, block_n, Din), lambda i: (0, i, 0)),
          pl.BlockSpec((Din, Hout), lambda i: (0, 0)),
          pl.BlockSpec((1, Hout), lambda i: (0, 0)),
      ],
      out_specs=pl.BlockSpec((block_n, Hout), lambda i: (i, 0)),
      out_shape=jax.ShapeDtypeStruct((N, Hout), jnp.float32),
  )(parts, W, b2d)


def _tc_loss(parts, W2, b2d, labels2d, n_valid):
  _, NP, _ = parts.shape
  C = W2.shape[1]

  def body(p_ref, w_ref, b_ref, l_ref, o_ref):
    x = jnp.dot(p_ref[0] + p_ref[1], w_ref[...],
                preferred_element_type=jnp.float32) + b_ref[...]
    m = jnp.max(x, axis=1, keepdims=True)
    lse = jnp.log(jnp.sum(jnp.exp(x - m), axis=1, keepdims=True)) + m
    ids = lax.broadcasted_iota(jnp.int32, (NP, C), 1)
    picked = jnp.sum(jnp.where(ids == l_ref[...], x, 0.0), axis=1,
                     keepdims=True)
    rows = lax.broadcasted_iota(jnp.int32, (NP, 1), 0)
    nll = jnp.where(rows < n_valid, lse - picked, 0.0)
    o_ref[...] = jnp.sum(nll, keepdims=True) / n_valid

  out = pl.pallas_call(
      body,
      out_shape=jax.ShapeDtypeStruct((1, 1), jnp.float32),
  )(parts, W2, b2d, labels2d)
  return out[0, 0]


@jax.jit
def kernel(features, edge_index, edge_weight, labels, W0, b0, W1, b1, W2, b2):
  N = features.shape[0]
  E = edge_weight.shape[0]
  NP = NS * 128 * -(--(-N // NS) // 128)

  n0 = NS * NB0 * B_E
  total = n0 + NS * NB1 * B_E
  assert total >= E
  pad = total - E

  def chunk(x):
    xp = jnp.pad(x, (0, pad))
    a0 = xp[:n0].reshape(NS, NB0, B_E)
    a1 = jnp.pad(xp[n0:].reshape(NS, NB1, B_E),
                 ((0, 0), (0, NB0 - NB1), (0, 0)))
    return jnp.stack([a0, a1], axis=1).reshape(NW, NB0, B_E)

  src3 = chunk(edge_index[0])
  dst3 = chunk(edge_index[1])
  w3 = chunk(edge_weight)

  labels_p = jnp.pad(labels.astype(jnp.int32), (0, NP - N)).reshape(-1, 1)

  a0 = _seg_sum(features, src3, dst3, w3, NP)
  h1 = _tc_layer(a0, W0, b0.reshape(1, -1), 2048)
  a1 = _seg_sum(h1, src3, dst3, w3, NP)
  h2 = _tc_layer(a1, W1, b1.reshape(1, -1), 2048)
  a2 = _seg_sum(h2, src3, dst3, w3, NP)
  return _tc_loss(a2, W2, b2.reshape(1, -1), labels_p, N)


# speedup vs baseline: 1.0009x; 1.0009x over previous
import jax
import jax.numpy as jnp
from jax import lax
from jax.experimental import pallas as pl
from jax.experimental.pallas import tpu as pltpu
from jax.experimental.pallas import tpu_sc as plsc

NC = 2
NS = 16
NW = NC * NS
L = 16
B_E = 128
NB0 = 120
NB1 = 40


def _seg_sum(h, src3, dst3, w3, np_rows):
  _, D = h.shape
  NB = src3.shape[1]
  CH = 8
  assert D % L == 0 and NB % CH == 0
  assert NB == NB0 and NB1 % CH == 0
  RPT = np_rows // NS
  assert RPT % B_E == 0

  mesh = plsc.VectorSubcoreMesh(core_axis_name="c", subcore_axis_name="s")

  def body(h_hbm, src_hbm, dst_hbm, w_hbm, out_hbm,
           src_v, dst_v, w_v, rows_a, rows_b, acc, sem_a, sem_b,
           sem_sa, sem_sb):
    cid = lax.axis_index("c")
    sid = lax.axis_index("s")
    wid = sid * NC + cid

    with jax.named_scope("zero_acc"):
      z16 = jnp.zeros((L,), jnp.float32)

      @pl.loop(0, B_E)
      def _(i):
        for c in range(D // L):
          rows_a[i, pl.ds(c * L, L)] = z16

      for k in range(RPT // B_E):
        pltpu.sync_copy(rows_a, acc.at[pl.ds(sid * RPT + k * B_E, B_E)])

      plsc.subcore_barrier()

    ng = jnp.where(cid == 0, NB0 // CH, NB1 // CH)

    def scale(rows, j):
      @pl.loop(0, B_E // L)
      def _(rb):
        wchunk = w_v[j, pl.ds(rb * L, L)]
        for i in range(L):
          wv = jnp.full((L,), wchunk[i])
          r = rb * L + i
          for c in range(D // L):
            sl = pl.ds(c * L, L)
            rows[r, sl] = rows[r, sl] * wv

    @pl.loop(0, ng)
    def _(g):
     with jax.named_scope("edge_group"):
       pltpu.sync_copy(src_hbm.at[wid, pl.ds(g * CH, CH)], src_v)
       pltpu.sync_copy(dst_hbm.at[wid, pl.ds(g * CH, CH)], dst_v)
       pltpu.sync_copy(w_hbm.at[wid, pl.ds(g * CH, CH)], w_v)

       pltpu.async_copy(h_hbm.at[src_v.at[0]], rows_a, sem_a)
       pltpu.async_copy(h_hbm.at[src_v.at[1]], rows_b, sem_b)

       @pl.loop(0, CH, step=2)
       def _(j):
         for (buf, sem, ssem, jj) in ((rows_a, sem_a, sem_sa, j),
                                      (rows_b, sem_b, sem_sb, j + 1)):
           pltpu.make_async_copy(h_hbm.at[src_v.at[jj]], buf, sem).wait()
           scale(buf, jj)
           pltpu.async_copy(buf, acc.at[dst_v.at[jj]], ssem, add=True)

         for (buf, sem, ssem, jj) in ((rows_a, sem_a, sem_sa, j),
                                      (rows_b, sem_b, sem_sb, j + 1)):
           pltpu.make_async_copy(buf, acc.at[dst_v.at[jj]], ssem).wait()

           @pl.when(jj + 2 < CH)
           def _():
             pltpu.async_copy(h_hbm.at[src_v.at[jj + 2]], buf, sem)

    with jax.named_scope("copy_out"):
      plsc.subcore_barrier()
      pltpu.sync_copy(acc.at[pl.ds(sid * RPT, RPT)],
                      out_hbm.at[cid, pl.ds(sid * RPT, RPT)])

  kern = pl.kernel(
      body,
      out_type=jax.ShapeDtypeStruct((NC, np_rows, D), jnp.float32),
      mesh=mesh,
      scratch_types=[
          pltpu.VMEM((CH, B_E), jnp.int32),
          pltpu.VMEM((CH, B_E), jnp.int32),
          pltpu.VMEM((CH, B_E), jnp.float32),
          pltpu.VMEM((B_E, D), jnp.float32),
          pltpu.VMEM((B_E, D), jnp.float32),
          pltpu.VMEM_SHARED((np_rows, D), jnp.float32),
          pltpu.SemaphoreType.DMA,
          pltpu.SemaphoreType.DMA,
          pltpu.SemaphoreType.DMA,
          pltpu.SemaphoreType.DMA,
      ],
  )
  return kern(h, src3, dst3, w3)


def _tc_layer(parts, W, b2d, block_n):
  _, N, Din = parts.shape
  Hout = W.shape[1]
  assert N % block_n == 0

  def body(p_ref, w_ref, b_ref, o_ref):
    x = p_ref[0] + p_ref[1]
    y = jnp.dot(x, w_ref[...], preferred_element_type=jnp.float32) + b_ref[...]
    o_ref[...] = jnp.maximum(y, 0.0)

  return pl.pallas_call(
      body,
      grid=(N // block_n,),
      in_specs=[
          pl.BlockSpec((NC, block_n, Din), lambda i: (0, i, 0)),
          pl.BlockSpec((Din, Hout), lambda i: (0, 0)),
          pl.BlockSpec((1, Hout), lambda i: (0, 0)),
      ],
      out_specs=pl.BlockSpec((block_n, Hout), lambda i: (i, 0)),
      out_shape=jax.ShapeDtypeStruct((N, Hout), jnp.float32),
  )(parts, W, b2d)


def _tc_loss(parts, W2, b2d, labels2d, n_valid):
  _, NP, _ = parts.shape
  C = W2.shape[1]

  def body(p_ref, w_ref, b_ref, l_ref, o_ref):
    x = jnp.dot(p_ref[0] + p_ref[1], w_ref[...],
                preferred_element_type=jnp.float32) + b_ref[...]
    m = jnp.max(x, axis=1, keepdims=True)
    lse = jnp.log(jnp.sum(jnp.exp(x - m), axis=1, keepdims=True)) + m
    ids = lax.broadcasted_iota(jnp.int32, (NP, C), 1)
    picked = jnp.sum(jnp.where(ids == l_ref[...], x, 0.0), axis=1,
                     keepdims=True)
    rows = lax.broadcasted_iota(jnp.int32, (NP, 1), 0)
    nll = jnp.where(rows < n_valid, lse - picked, 0.0)
    o_ref[...] = jnp.sum(nll, keepdims=True) / n_valid

  out = pl.pallas_call(
      body,
      out_shape=jax.ShapeDtypeStruct((1, 1), jnp.float32),
  )(parts, W2, b2d, labels2d)
  return out[0, 0]


@jax.jit
def kernel(features, edge_index, edge_weight, labels, W0, b0, W1, b1, W2, b2):
  N = features.shape[0]
  E = edge_weight.shape[0]
  NP = NS * 128 * -(--(-N // NS) // 128)

  n0 = NS * NB0 * B_E
  total = n0 + NS * NB1 * B_E
  assert total >= E
  pad = total - E

  def chunk(x):
    xp = jnp.pad(x, (0, pad))
    a0 = xp[:n0].reshape(NS, NB0, B_E)
    a1 = jnp.pad(xp[n0:].reshape(NS, NB1, B_E),
                 ((0, 0), (0, NB0 - NB1), (0, 0)))
    return jnp.stack([a0, a1], axis=1).reshape(NW, NB0, B_E)

  src3 = chunk(edge_index[0])
  dst3 = chunk(edge_index[1])
  w3 = chunk(edge_weight)

  labels_p = jnp.pad(labels.astype(jnp.int32), (0, NP - N)).reshape(-1, 1)

  a0 = _seg_sum(features, src3, dst3, w3, NP)
  h1 = _tc_layer(a0, W0, b0.reshape(1, -1), 2048)
  a1 = _seg_sum(h1, src3, dst3, w3, NP)
  h2 = _tc_layer(a1, W1, b1.reshape(1, -1), 2048)
  a2 = _seg_sum(h2, src3, dst3, w3, NP)
  return _tc_loss(a2, W2, b2.reshape(1, -1), labels_p, N)


# speedup vs baseline: 2.5188x; 2.5164x over previous
import jax
import jax.numpy as jnp
from jax import lax
from jax.experimental import pallas as pl
from jax.experimental.pallas import tpu as pltpu
from jax.experimental.pallas import tpu_sc as plsc

NC = 2
NS = 16
NW = NC * NS
L = 16
B_E = 128


def _seg_sum(h, src3, dst3, w3, np_rows):
  _, D = h.shape
  NB = src3.shape[1]
  CH = 16
  assert D % L == 0 and NB % CH == 0
  RPT = np_rows // NS
  assert RPT % B_E == 0

  mesh = plsc.VectorSubcoreMesh(core_axis_name="c", subcore_axis_name="s")

  def body(h_hbm, src_hbm, dst_hbm, w_hbm, out_hbm,
           src_v, dst_v, w_v, rows_a, rows_b, acc, sem_a, sem_b,
           sem_sa, sem_sb):
    cid = lax.axis_index("c")
    sid = lax.axis_index("s")
    wid = sid * NC + cid

    with jax.named_scope("zero_acc"):
      z16 = jnp.zeros((L,), jnp.float32)

      @pl.loop(0, B_E)
      def _(i):
        for c in range(D // L):
          rows_a[i, pl.ds(c * L, L)] = z16

      for k in range(RPT // B_E):
        pltpu.sync_copy(rows_a, acc.at[pl.ds(sid * RPT + k * B_E, B_E)])

      plsc.subcore_barrier()

    def scale(rows, j):
      @pl.loop(0, B_E // L)
      def _(rb):
        wchunk = w_v[j, pl.ds(rb * L, L)]
        for i in range(L):
          wv = jnp.full((L,), wchunk[i])
          r = rb * L + i
          for c in range(D // L):
            sl = pl.ds(c * L, L)
            rows[r, sl] = rows[r, sl] * wv

    @pl.loop(0, NB // CH)
    def _(g):
     with jax.named_scope("edge_group"):
       pltpu.sync_copy(src_hbm.at[wid, pl.ds(g * CH, CH)], src_v)
       pltpu.sync_copy(dst_hbm.at[wid, pl.ds(g * CH, CH)], dst_v)
       pltpu.sync_copy(w_hbm.at[wid, pl.ds(g * CH, CH)], w_v)

       pltpu.async_copy(h_hbm.at[src_v.at[0]], rows_a, sem_a)
       pltpu.async_copy(h_hbm.at[src_v.at[1]], rows_b, sem_b)

       @pl.loop(0, CH, step=2)
       def _(j):
         for (buf, sem, ssem, jj) in ((rows_a, sem_a, sem_sa, j),
                                      (rows_b, sem_b, sem_sb, j + 1)):
           pltpu.make_async_copy(h_hbm.at[src_v.at[jj]], buf, sem).wait()
           scale(buf, jj)
           pltpu.async_copy(buf, acc.at[dst_v.at[jj]], ssem, add=True)

         for (buf, sem, ssem, jj) in ((rows_a, sem_a, sem_sa, j),
                                      (rows_b, sem_b, sem_sb, j + 1)):
           pltpu.make_async_copy(buf, acc.at[dst_v.at[jj]], ssem).wait()

           @pl.when(jj + 2 < CH)
           def _():
             pltpu.async_copy(h_hbm.at[src_v.at[jj + 2]], buf, sem)

    with jax.named_scope("copy_out"):
      plsc.subcore_barrier()
      pltpu.sync_copy(acc.at[pl.ds(sid * RPT, RPT)],
                      out_hbm.at[cid, pl.ds(sid * RPT, RPT)])

  kern = pl.kernel(
      body,
      out_type=jax.ShapeDtypeStruct((NC, np_rows, D), jnp.float32),
      mesh=mesh,
      scratch_types=[
          pltpu.VMEM((CH, B_E), jnp.int32),
          pltpu.VMEM((CH, B_E), jnp.int32),
          pltpu.VMEM((CH, B_E), jnp.float32),
          pltpu.VMEM((B_E, D), jnp.float32),
          pltpu.VMEM((B_E, D), jnp.float32),
          pltpu.VMEM_SHARED((np_rows, D), jnp.float32),
          pltpu.SemaphoreType.DMA,
          pltpu.SemaphoreType.DMA,
          pltpu.SemaphoreType.DMA,
          pltpu.SemaphoreType.DMA,
      ],
  )
  return kern(h, src3, dst3, w3)


def _tc_layer(parts, W, b2d, block_n):
  _, N, Din = parts.shape
  Hout = W.shape[1]
  assert N % block_n == 0

  def body(p_ref, w_ref, b_ref, o_ref):
    x = p_ref[0] + p_ref[1]
    y = jnp.dot(x, w_ref[...], preferred_element_type=jnp.float32) + b_ref[...]
    o_ref[...] = jnp.maximum(y, 0.0)

  return pl.pallas_call(
      body,
      grid=(N // block_n,),
      in_specs=[
          pl.BlockSpec((NC, block_n, Din), lambda i: (0, i, 0)),
          pl.BlockSpec((Din, Hout), lambda i: (0, 0)),
          pl.BlockSpec((1, Hout), lambda i: (0, 0)),
      ],
      out_specs=pl.BlockSpec((block_n, Hout), lambda i: (i, 0)),
      out_shape=jax.ShapeDtypeStruct((N, Hout), jnp.float32),
  )(parts, W, b2d)


def _tc_loss(parts, W2, b2d, labels2d, n_valid):
  _, NP, _ = parts.shape
  C = W2.shape[1]

  def body(p_ref, w_ref, b_ref, l_ref, o_ref):
    x = jnp.dot(p_ref[0] + p_ref[1], w_ref[...],
                preferred_element_type=jnp.float32) + b_ref[...]
    m = jnp.max(x, axis=1, keepdims=True)
    lse = jnp.log(jnp.sum(jnp.exp(x - m), axis=1, keepdims=True)) + m
    ids = lax.broadcasted_iota(jnp.int32, (NP, C), 1)
    picked = jnp.sum(jnp.where(ids == l_ref[...], x, 0.0), axis=1,
                     keepdims=True)
    rows = lax.broadcasted_iota(jnp.int32, (NP, 1), 0)
    nll = jnp.where(rows < n_valid, lse - picked, 0.0)
    o_ref[...] = jnp.sum(nll, keepdims=True) / n_valid

  out = pl.pallas_call(
      body,
      out_shape=jax.ShapeDtypeStruct((1, 1), jnp.float32),
  )(parts, W2, b2d, labels2d)
  return out[0, 0]


@jax.jit
def kernel(features, edge_index, edge_weight, labels, W0, b0, W1, b1, W2, b2):
  N = features.shape[0]
  E = edge_weight.shape[0]
  NP = NS * 128 * -(--(-N // NS) // 128)

  NB = -(-(-(-E // NW)) // B_E)
  NB += NB % 2
  total = NW * NB * B_E
  pad = total - E
  pad_idx = jnp.arange(pad, dtype=jnp.int32) % jnp.int32(B_E)
  src3 = jnp.concatenate([edge_index[0], pad_idx]).reshape(NW, NB, B_E)
  dst3 = jnp.concatenate([edge_index[1], pad_idx]).reshape(NW, NB, B_E)
  w3 = jnp.pad(edge_weight, (0, pad)).reshape(NW, NB, B_E)

  labels_p = jnp.pad(labels.astype(jnp.int32), (0, NP - N)).reshape(-1, 1)

  a0 = _seg_sum(features, src3, dst3, w3, NP)
  h1 = _tc_layer(a0, W0, b0.reshape(1, -1), 2048)
  a1 = _seg_sum(h1, src3, dst3, w3, NP)
  h2 = _tc_layer(a1, W1, b1.reshape(1, -1), 2048)
  a2 = _seg_sum(h2, src3, dst3, w3, NP)
  return _tc_loss(a2, W2, b2.reshape(1, -1), labels_p, N)


# speedup vs baseline: 2.5476x; 1.0115x over previous
import jax
import jax.numpy as jnp
from jax import lax
from jax.experimental import pallas as pl
from jax.experimental.pallas import tpu as pltpu
from jax.experimental.pallas import tpu_sc as plsc

NC = 2
NS = 16
NW = NC * NS
L = 16
B_E = 128


def _seg_sum(h, src3, dst3, w3, np_rows):
  _, D = h.shape
  NB = src3.shape[1]
  CH = 16
  assert D % L == 0 and NB % CH == 0
  RPT = np_rows // NS
  assert RPT % B_E == 0

  mesh = plsc.VectorSubcoreMesh(core_axis_name="c", subcore_axis_name="s")

  def body(h_hbm, src_hbm, dst_hbm, w_hbm, out_hbm,
           src_v, dst_v, w_v, rows_a, rows_b, acc, sem_a, sem_b,
           sem_sa, sem_sb):
    cid = lax.axis_index("c")
    sid = lax.axis_index("s")
    wid = sid * NC + cid

    with jax.named_scope("zero_acc"):
      z16 = jnp.zeros((L,), jnp.float32)

      @pl.loop(0, B_E)
      def _(i):
        for c in range(D // L):
          rows_a[i, pl.ds(c * L, L)] = z16

      for k in range(RPT // B_E):
        pltpu.sync_copy(rows_a, acc.at[pl.ds(sid * RPT + k * B_E, B_E)])

      plsc.subcore_barrier()

    def scale(rows, j):
      @pl.loop(0, B_E // L, unroll=2)
      def _(rb):
        wchunk = w_v[j, pl.ds(rb * L, L)]
        for i in range(L):
          wv = jnp.full((L,), wchunk[i])
          r = rb * L + i
          for c in range(D // L):
            sl = pl.ds(c * L, L)
            rows[r, sl] = rows[r, sl] * wv

    @pl.loop(0, NB // CH)
    def _(g):
      pltpu.sync_copy(src_hbm.at[wid, pl.ds(g * CH, CH)], src_v)
      pltpu.sync_copy(dst_hbm.at[wid, pl.ds(g * CH, CH)], dst_v)
      pltpu.sync_copy(w_hbm.at[wid, pl.ds(g * CH, CH)], w_v)

      pltpu.async_copy(h_hbm.at[src_v.at[0]], rows_a, sem_a)
      pltpu.async_copy(h_hbm.at[src_v.at[1]], rows_b, sem_b)

      @pl.loop(0, CH, step=2)
      def _(j):
        for (buf, sem, ssem, jj) in ((rows_a, sem_a, sem_sa, j),
                                     (rows_b, sem_b, sem_sb, j + 1)):
          pltpu.make_async_copy(h_hbm.at[src_v.at[jj]], buf, sem).wait()
          scale(buf, jj)
          pltpu.async_copy(buf, acc.at[dst_v.at[jj]], ssem, add=True)

        for (buf, sem, ssem, jj) in ((rows_a, sem_a, sem_sa, j),
                                     (rows_b, sem_b, sem_sb, j + 1)):
          pltpu.make_async_copy(buf, acc.at[dst_v.at[jj]], ssem).wait()

          @pl.when(jj + 2 < CH)
          def _():
            pltpu.async_copy(h_hbm.at[src_v.at[jj + 2]], buf, sem)

    with jax.named_scope("copy_out"):
      plsc.subcore_barrier()
      pltpu.sync_copy(acc.at[pl.ds(sid * RPT, RPT)],
                      out_hbm.at[cid, pl.ds(sid * RPT, RPT)])

  kern = pl.kernel(
      body,
      out_type=jax.ShapeDtypeStruct((NC, np_rows, D), jnp.float32),
      mesh=mesh,
      scratch_types=[
          pltpu.VMEM((CH, B_E), jnp.int32),
          pltpu.VMEM((CH, B_E), jnp.int32),
          pltpu.VMEM((CH, B_E), jnp.float32),
          pltpu.VMEM((B_E, D), jnp.float32),
          pltpu.VMEM((B_E, D), jnp.float32),
          pltpu.VMEM_SHARED((np_rows, D), jnp.float32),
          pltpu.SemaphoreType.DMA,
          pltpu.SemaphoreType.DMA,
          pltpu.SemaphoreType.DMA,
          pltpu.SemaphoreType.DMA,
      ],
  )
  return kern(h, src3, dst3, w3)


def _tc_layer(parts, W, b2d, block_n):
  _, N, Din = parts.shape
  Hout = W.shape[1]
  assert N % block_n == 0

  def body(p_ref, w_ref, b_ref, o_ref):
    x = p_ref[0] + p_ref[1]
    y = jnp.dot(x, w_ref[...], preferred_element_type=jnp.float32) + b_ref[...]
    o_ref[...] = jnp.maximum(y, 0.0)

  return pl.pallas_call(
      body,
      grid=(N // block_n,),
      in_specs=[
          pl.BlockSpec((NC, block_n, Din), lambda i: (0, i, 0)),
          pl.BlockSpec((Din, Hout), lambda i: (0, 0)),
          pl.BlockSpec((1, Hout), lambda i: (0, 0)),
      ],
      out_specs=pl.BlockSpec((block_n, Hout), lambda i: (i, 0)),
      out_shape=jax.ShapeDtypeStruct((N, Hout), jnp.float32),
  )(parts, W, b2d)


def _tc_loss(parts, W2, b2d, labels2d, n_valid):
  _, NP, _ = parts.shape
  C = W2.shape[1]

  def body(p_ref, w_ref, b_ref, l_ref, o_ref):
    x = jnp.dot(p_ref[0] + p_ref[1], w_ref[...],
                preferred_element_type=jnp.float32) + b_ref[...]
    m = jnp.max(x, axis=1, keepdims=True)
    lse = jnp.log(jnp.sum(jnp.exp(x - m), axis=1, keepdims=True)) + m
    ids = lax.broadcasted_iota(jnp.int32, (NP, C), 1)
    picked = jnp.sum(jnp.where(ids == l_ref[...], x, 0.0), axis=1,
                     keepdims=True)
    rows = lax.broadcasted_iota(jnp.int32, (NP, 1), 0)
    nll = jnp.where(rows < n_valid, lse - picked, 0.0)
    o_ref[...] = jnp.sum(nll, keepdims=True) / n_valid

  out = pl.pallas_call(
      body,
      out_shape=jax.ShapeDtypeStruct((1, 1), jnp.float32),
  )(parts, W2, b2d, labels2d)
  return out[0, 0]


@jax.jit
def kernel(features, edge_index, edge_weight, labels, W0, b0, W1, b1, W2, b2):
  N = features.shape[0]
  E = edge_weight.shape[0]
  NP = NS * 128 * -(--(-N // NS) // 128)

  NB = -(-(-(-E // NW)) // B_E)
  NB += NB % 2
  total = NW * NB * B_E
  pad = total - E
  pad_idx = jnp.arange(pad, dtype=jnp.int32) % jnp.int32(B_E)
  src3 = jnp.concatenate([edge_index[0], pad_idx]).reshape(NW, NB, B_E)
  dst3 = jnp.concatenate([edge_index[1], pad_idx]).reshape(NW, NB, B_E)
  w3 = jnp.pad(edge_weight, (0, pad)).reshape(NW, NB, B_E)

  labels_p = jnp.pad(labels.astype(jnp.int32), (0, NP - N)).reshape(-1, 1)

  a0 = _seg_sum(features, src3, dst3, w3, NP)
  h1 = _tc_layer(a0, W0, b0.reshape(1, -1), 2048)
  a1 = _seg_sum(h1, src3, dst3, w3, NP)
  h2 = _tc_layer(a1, W1, b1.reshape(1, -1), 2048)
  a2 = _seg_sum(h2, src3, dst3, w3, NP)
  return _tc_loss(a2, W2, b2.reshape(1, -1), labels_p, N)


# speedup vs baseline: 2.8189x; 1.1065x over previous
import jax
import jax.numpy as jnp
from jax import lax
from jax.experimental import pallas as pl
from jax.experimental.pallas import tpu as pltpu
from jax.experimental.pallas import tpu_sc as plsc

NC = 2
NS = 16
NW = NC * NS
L = 16
B_E = 96
CH = 18
NBUF = 3


def _seg_sum(h, src4, dst4, w4, np_rows):
  _, D = h.shape
  NG = src4.shape[1]
  assert D % L == 0 and NG >= 2 and CH % NBUF == 0
  RPT = np_rows // NS
  ZCH = 64
  assert RPT % ZCH == 0 and ZCH <= B_E

  mesh = plsc.VectorSubcoreMesh(core_axis_name="c", subcore_axis_name="s")

  def body(h_hbm, src_hbm, dst_hbm, w_hbm, out_hbm,
           src_v, dst_v, w_v, rows, acc, gsem, ssem):
    cid = lax.axis_index("c")
    sid = lax.axis_index("s")
    wid = sid * NC + cid

    with jax.named_scope("zero_acc"):
      z16 = jnp.zeros((L,), jnp.float32)

      @pl.loop(0, ZCH)
      def _(i):
        for c in range(D // L):
          rows[0][i, pl.ds(c * L, L)] = z16

      for k in range(RPT // ZCH):
        pltpu.sync_copy(rows[0].at[pl.ds(0, ZCH)],
                        acc.at[pl.ds(sid * RPT + k * ZCH, ZCH)])

      plsc.subcore_barrier()

    def scale(buf, j):
      @pl.loop(0, B_E // L, unroll=2)
      def _(rb):
        wchunk = w_v[j, pl.ds(rb * L, L)]
        for i in range(L):
          wv = jnp.full((L,), wchunk[i])
          r = rb * L + i
          for c in range(D // L):
            sl = pl.ds(c * L, L)
            buf[r, sl] = buf[r, sl] * wv

    def stage_sync(g):
      pltpu.sync_copy(src_hbm.at[wid, g], src_v)
      pltpu.sync_copy(dst_hbm.at[wid, g], dst_v)
      pltpu.sync_copy(w_hbm.at[wid, g], w_v)

    def gather(j, b):
      pltpu.async_copy(h_hbm.at[src_v.at[j]], rows[b], gsem[b])

    def gather_wait(j, b):
      pltpu.make_async_copy(h_hbm.at[src_v.at[j]], rows[b], gsem[b]).wait()

    def scatter(j, b):
      pltpu.async_copy(rows[b], acc.at[dst_v.at[j]], ssem[b], add=True)

    def scatter_wait(b):
      pltpu.make_async_copy(rows[b], acc.at[dst_v.at[0]], ssem[b]).wait()

    def run_group():
      for jb in (0, 1):
        gather(jb, jb)

      @pl.loop(0, CH, step=NBUF)
      def _(j):
        for k in range(NBUF):
          b = k
          gather_wait(j + k, b)
          scale(rows[b], j + k)
          scatter(j + k, b)
          tb = (k + 2) % NBUF

          @pl.when(j + k + 2 < CH)
          def _():
            if k == 0:
              @pl.when(j > 0)
              def _():
                scatter_wait(tb)
            else:
              scatter_wait(tb)
            gather(j + k + 2, tb)

    stage_sync(0)
    run_group()

    @pl.loop(1, NG)
    def _(g):
      for jb in range(NBUF):
        scatter_wait(jb)
      stage_sync(g)
      run_group()

    for jb in range(NBUF):
      scatter_wait(jb)

    with jax.named_scope("copy_out"):
      plsc.subcore_barrier()
      pltpu.sync_copy(acc.at[pl.ds(sid * RPT, RPT)],
                      out_hbm.at[cid, pl.ds(sid * RPT, RPT)])

  kern = pl.kernel(
      body,
      out_type=jax.ShapeDtypeStruct((NC, np_rows, D), jnp.float32),
      mesh=mesh,
      scratch_types=[
          pltpu.VMEM((CH, B_E), jnp.int32),
          pltpu.VMEM((CH, B_E), jnp.int32),
          pltpu.VMEM((CH, B_E), jnp.float32),
          [pltpu.VMEM((B_E, D), jnp.float32) for _ in range(NBUF)],
          pltpu.VMEM_SHARED((np_rows, D), jnp.float32),
          [pltpu.SemaphoreType.DMA for _ in range(NBUF)],
          [pltpu.SemaphoreType.DMA for _ in range(NBUF)],
      ],
  )
  return kern(h, src4, dst4, w4)


def _tc_layer(parts, W, b2d, block_n):
  _, N, Din = parts.shape
  Hout = W.shape[1]
  assert N % block_n == 0

  def body(p_ref, w_ref, b_ref, o_ref):
    x = p_ref[0] + p_ref[1]
    y = jnp.dot(x, w_ref[...], preferred_element_type=jnp.float32) + b_ref[...]
    o_ref[...] = jnp.maximum(y, 0.0)

  return pl.pallas_call(
      body,
      grid=(N // block_n,),
      in_specs=[
          pl.BlockSpec((NC, block_n, Din), lambda i: (0, i, 0)),
          pl.BlockSpec((Din, Hout), lambda i: (0, 0)),
          pl.BlockSpec((1, Hout), lambda i: (0, 0)),
      ],
      out_specs=pl.BlockSpec((block_n, Hout), lambda i: (i, 0)),
      out_shape=jax.ShapeDtypeStruct((N, Hout), jnp.float32),
  )(parts, W, b2d)


def _tc_loss(parts, W2, b2d, labels2d, n_valid):
  _, NP, _ = parts.shape
  C = W2.shape[1]

  def body(p_ref, w_ref, b_ref, l_ref, o_ref):
    x = jnp.dot(p_ref[0] + p_ref[1], w_ref[...],
                preferred_element_type=jnp.float32) + b_ref[...]
    m = jnp.max(x, axis=1, keepdims=True)
    lse = jnp.log(jnp.sum(jnp.exp(x - m), axis=1, keepdims=True)) + m
    ids = lax.broadcasted_iota(jnp.int32, (NP, C), 1)
    picked = jnp.sum(jnp.where(ids == l_ref[...], x, 0.0), axis=1,
                     keepdims=True)
    rows = lax.broadcasted_iota(jnp.int32, (NP, 1), 0)
    nll = jnp.where(rows < n_valid, lse - picked, 0.0)
    o_ref[...] = jnp.sum(nll, keepdims=True) / n_valid

  out = pl.pallas_call(
      body,
      out_shape=jax.ShapeDtypeStruct((1, 1), jnp.float32),
  )(parts, W2, b2d, labels2d)
  return out[0, 0]


@jax.jit
def kernel(features, edge_index, edge_weight, labels, W0, b0, W1, b1, W2, b2):
  N = features.shape[0]
  E = edge_weight.shape[0]
  NP = NS * 128 * -(--(-N // NS) // 128)

  GE = CH * B_E
  NG = -(-(-(-E // NW)) // GE)
  total = NW * NG * GE
  pad = total - E
  pad_idx = jnp.arange(pad, dtype=jnp.int32) % jnp.int32(B_E)
  src4 = jnp.concatenate([edge_index[0], pad_idx]).reshape(NW, NG, CH, B_E)
  dst4 = jnp.concatenate([edge_index[1], pad_idx]).reshape(NW, NG, CH, B_E)
  w4 = jnp.pad(edge_weight, (0, pad)).reshape(NW, NG, CH, B_E)

  labels_p = jnp.pad(labels.astype(jnp.int32), (0, NP - N)).reshape(-1, 1)

  a0 = _seg_sum(features, src4, dst4, w4, NP)
  h1 = _tc_layer(a0, W0, b0.reshape(1, -1), 2048)
  a1 = _seg_sum(h1, src4, dst4, w4, NP)
  h2 = _tc_layer(a1, W1, b1.reshape(1, -1), 2048)
  a2 = _seg_sum(h2, src4, dst4, w4, NP)
  return _tc_loss(a2, W2, b2.reshape(1, -1), labels_p, N)


# speedup vs baseline: 2.9291x; 1.0391x over previous
import jax
import jax.numpy as jnp
from jax import lax
from jax.experimental import pallas as pl
from jax.experimental.pallas import tpu as pltpu
from jax.experimental.pallas import tpu_sc as plsc

NC = 2
NS = 16
NW = NC * NS
L = 16
B_E = 96
CH = 21
NBUF = 3


def _seg_sum(h, src4, dst4, w4, np_rows):
  _, D = h.shape
  NG = src4.shape[1]
  assert D % L == 0 and NG >= 2 and CH % NBUF == 0
  RPT = np_rows // NS
  ZCH = 64
  assert RPT % ZCH == 0 and ZCH <= B_E

  mesh = plsc.VectorSubcoreMesh(core_axis_name="c", subcore_axis_name="s")

  def body(h_hbm, src_hbm, dst_hbm, w_hbm, out_hbm,
           src_v, dst_v, w_v, rows, acc, gsem, ssem):
    cid = lax.axis_index("c")
    sid = lax.axis_index("s")
    wid = sid * NC + cid

    z16 = jnp.zeros((L,), jnp.float32)

    @pl.loop(0, ZCH)
    def _(i):
      for c in range(D // L):
        rows[0][i, pl.ds(c * L, L)] = z16

    for k in range(RPT // ZCH):
      pltpu.sync_copy(rows[0].at[pl.ds(0, ZCH)],
                      acc.at[pl.ds(sid * RPT + k * ZCH, ZCH)])

    plsc.subcore_barrier()

    def scale(buf, j):
      @pl.loop(0, B_E // L, unroll=2)
      def _(rb):
        wchunk = w_v[j, pl.ds(rb * L, L)]
        for i in range(L):
          wv = jnp.full((L,), wchunk[i])
          r = rb * L + i
          for c in range(D // L):
            sl = pl.ds(c * L, L)
            buf[r, sl] = buf[r, sl] * wv

    def stage_sync(g):
      pltpu.sync_copy(src_hbm.at[wid, g], src_v)
      pltpu.sync_copy(dst_hbm.at[wid, g], dst_v)
      pltpu.sync_copy(w_hbm.at[wid, g], w_v)

    def gather(j, b):
      pltpu.async_copy(h_hbm.at[src_v.at[j]], rows[b], gsem[b])

    def gather_wait(j, b):
      pltpu.make_async_copy(h_hbm.at[src_v.at[j]], rows[b], gsem[b]).wait()

    def scatter(j, b):
      pltpu.async_copy(rows[b], acc.at[dst_v.at[j]], ssem[b], add=True)

    def scatter_wait(b):
      pltpu.make_async_copy(rows[b], acc.at[dst_v.at[0]], ssem[b]).wait()

    def run_group():
      for jb in (0, 1):
        gather(jb, jb)

      @pl.loop(0, CH, step=NBUF)
      def _(j):
        for k in range(NBUF):
          b = k
          gather_wait(j + k, b)
          scale(rows[b], j + k)
          scatter(j + k, b)
          tb = (k + 2) % NBUF

          @pl.when(j + k + 2 < CH)
          def _():
            if k == 0:
              @pl.when(j > 0)
              def _():
                scatter_wait(tb)
            else:
              scatter_wait(tb)
            gather(j + k + 2, tb)

    stage_sync(0)
    run_group()

    @pl.loop(1, NG)
    def _(g):
      for jb in range(NBUF):
        scatter_wait(jb)
      stage_sync(g)
      run_group()

    for jb in range(NBUF):
      scatter_wait(jb)

    plsc.subcore_barrier()
    pltpu.sync_copy(acc.at[pl.ds(sid * RPT, RPT)],
                    out_hbm.at[cid, pl.ds(sid * RPT, RPT)])

  kern = pl.kernel(
      body,
      out_type=jax.ShapeDtypeStruct((NC, np_rows, D), jnp.float32),
      mesh=mesh,
      scratch_types=[
          pltpu.VMEM((CH, B_E), jnp.int32),
          pltpu.VMEM((CH, B_E), jnp.int32),
          pltpu.VMEM((CH, B_E), jnp.float32),
          [pltpu.VMEM((B_E, D), jnp.float32) for _ in range(NBUF)],
          pltpu.VMEM_SHARED((np_rows, D), jnp.float32),
          [pltpu.SemaphoreType.DMA for _ in range(NBUF)],
          [pltpu.SemaphoreType.DMA for _ in range(NBUF)],
      ],
  )
  return kern(h, src4, dst4, w4)


def _tc_layer(parts, W, b2d, block_n):
  _, N, Din = parts.shape
  Hout = W.shape[1]
  assert N % block_n == 0

  def body(p_ref, w_ref, b_ref, o_ref):
    x = p_ref[0] + p_ref[1]
    y = jnp.dot(x, w_ref[...], preferred_element_type=jnp.float32) + b_ref[...]
    o_ref[...] = jnp.maximum(y, 0.0)

  return pl.pallas_call(
      body,
      grid=(N // block_n,),
      in_specs=[
          pl.BlockSpec((NC, block_n, Din), lambda i: (0, i, 0)),
          pl.BlockSpec((Din, Hout), lambda i: (0, 0)),
          pl.BlockSpec((1, Hout), lambda i: (0, 0)),
      ],
      out_specs=pl.BlockSpec((block_n, Hout), lambda i: (i, 0)),
      out_shape=jax.ShapeDtypeStruct((N, Hout), jnp.float32),
  )(parts, W, b2d)


def _tc_loss(parts, W2, b2d, labels2d, n_valid):
  _, NP, _ = parts.shape
  C = W2.shape[1]

  def body(p_ref, w_ref, b_ref, l_ref, o_ref):
    x = jnp.dot(p_ref[0] + p_ref[1], w_ref[...],
                preferred_element_type=jnp.float32) + b_ref[...]
    m = jnp.max(x, axis=1, keepdims=True)
    lse = jnp.log(jnp.sum(jnp.exp(x - m), axis=1, keepdims=True)) + m
    ids = lax.broadcasted_iota(jnp.int32, (NP, C), 1)
    picked = jnp.sum(jnp.where(ids == l_ref[...], x, 0.0), axis=1,
                     keepdims=True)
    rows = lax.broadcasted_iota(jnp.int32, (NP, 1), 0)
    nll = jnp.where(rows < n_valid, lse - picked, 0.0)
    o_ref[...] = jnp.sum(nll, keepdims=True) / n_valid

  out = pl.pallas_call(
      body,
      out_shape=jax.ShapeDtypeStruct((1, 1), jnp.float32),
  )(parts, W2, b2d, labels2d)
  return out[0, 0]


@jax.jit
def kernel(features, edge_index, edge_weight, labels, W0, b0, W1, b1, W2, b2):
  N = features.shape[0]
  E = edge_weight.shape[0]
  NP = NS * 128 * -(--(-N // NS) // 128)

  GE = CH * B_E
  NG = -(-(-(-E // NW)) // GE)
  total = NW * NG * GE
  pad = total - E
  pad_idx = jnp.arange(pad, dtype=jnp.int32) % jnp.int32(B_E)
  src4 = jnp.concatenate([edge_index[0], pad_idx]).reshape(NW, NG, CH, B_E)
  dst4 = jnp.concatenate([edge_index[1], pad_idx]).reshape(NW, NG, CH, B_E)
  w4 = jnp.pad(edge_weight, (0, pad)).reshape(NW, NG, CH, B_E)

  labels_p = jnp.pad(labels.astype(jnp.int32), (0, NP - N)).reshape(-1, 1)

  a0 = _seg_sum(features, src4, dst4, w4, NP)
  h1 = _tc_layer(a0, W0, b0.reshape(1, -1), 2048)
  a1 = _seg_sum(h1, src4, dst4, w4, NP)
  h2 = _tc_layer(a1, W1, b1.reshape(1, -1), 2048)
  a2 = _seg_sum(h2, src4, dst4, w4, NP)
  return _tc_loss(a2, W2, b2.reshape(1, -1), labels_p, N)
